# baseline (speedup 1.0000x reference)
import jax
import jax.numpy as jnp
from jax.experimental import pallas as pl

_C = 512
_SP = 128


def _body(row_ref, col_ref, adj_top_ref, adj_bot_ref, se_ref, out_ref):
    N = adj_top_ref.shape[2]
    H = adj_top_ref.shape[1]
    bj = col_ref[0, 0:5, :]
    colm = col_ref[0, 5:6, :]

    def make_chunk(adj_ref, row_base):
        def chunk(c, t1):
            r0 = row_base + c * _C
            ai = row_ref[0, pl.ds(r0, _C), 0:5]
            d2 = jax.lax.dot_general(
                ai, bj, (((1,), (0,)), ((), ())),
                precision=jax.lax.Precision.HIGHEST,
                preferred_element_type=jnp.float32,
            )
            d2 = jnp.maximum(d2, 1e-12)
            dist = d2 * jax.lax.rsqrt(d2)
            mi = row_ref[0, pl.ds(r0, _C), 5:6]
            w = adj_ref[0, pl.ds(c * _C, _C), :].astype(jnp.float32) * dist * mi
            return t1 + jnp.sum(w, axis=0, keepdims=True)

        return chunk

    t1 = jax.lax.fori_loop(
        0, H // _C, make_chunk(adj_top_ref, 0), jnp.zeros((1, N), jnp.float32)
    )
    t1 = jax.lax.fori_loop(0, H // _C, make_chunk(adj_bot_ref, H), t1)
    pair = jnp.sum(t1 * colm)

    ids = row_ref[0, :, 6:7]
    sp = jax.lax.broadcasted_iota(jnp.int32, (1, _SP), 1).astype(jnp.float32)
    oh = (ids == sp).astype(jnp.float32)
    atom = jnp.sum(oh * se_ref[0])

    out_ref[...] = jnp.full_like(out_ref, atom + pair)


def kernel(node_indices, positions, adjacency, mask, species_energy, pair_weight):
    B, N = node_indices.shape
    S = species_energy.shape[0]

    maskf = mask.astype(jnp.float32)
    idsf = jnp.where(mask, node_indices, -1).astype(jnp.float32)
    r2 = jnp.sum(positions * positions, axis=-1, keepdims=True)
    onesc = jnp.ones_like(r2)

    rowpack = jnp.concatenate(
        [positions, r2, onesc, maskf[:, :, None], idsf[:, :, None]], axis=-1
    )
    rowpack = jnp.pad(rowpack, ((0, 0), (0, 0), (0, 128 - 7)))

    pw = pair_weight.astype(jnp.float32)
    colpack = jnp.concatenate(
        [
            -2.0 * positions.transpose(0, 2, 1),
            onesc.transpose(0, 2, 1),
            r2.transpose(0, 2, 1) + 2e-4,
            pw * maskf[:, None, :],
            jnp.zeros((B, 2, N), jnp.float32),
        ],
        axis=1,
    )

    se_row = jnp.zeros((1, 1, _SP), jnp.float32).at[0, 0, :S].set(species_energy)

    out = pl.pallas_call(
        _body,
        grid=(B,),
        in_specs=[
            pl.BlockSpec((1, N, 128), lambda b: (b, 0, 0)),
            pl.BlockSpec((1, 8, N), lambda b: (b, 0, 0)),
            pl.BlockSpec((1, N // 2, N), lambda b: (b, 0, 0)),
            pl.BlockSpec((1, N // 2, N), lambda b: (b, 1, 0)),
            pl.BlockSpec((1, 1, _SP), lambda b: (0, 0, 0)),
        ],
        out_specs=pl.BlockSpec((1, 1, 128), lambda b: (b, 0, 0)),
        out_shape=jax.ShapeDtypeStruct((B, 1, 128), jnp.float32),
    )(rowpack, colpack, adjacency, adjacency, se_row)

    return out[:, 0, 0]


# speedup vs baseline: 2.0392x; 2.0392x over previous
import jax
import jax.numpy as jnp
from jax.experimental import pallas as pl

_C = 512
_SP = 128


def _body(row_ref, col_ref, adj_top_ref, adj_bot_ref, se_ref, out_ref):
    N = adj_top_ref.shape[2]
    H = adj_top_ref.shape[1]
    bj = col_ref[0, 0:5, :]
    colm = col_ref[0, 5:6, :]

    def make_chunk(adj_ref, row_base):
        def chunk(c, t1):
            r0 = row_base + c * _C
            ai = row_ref[0, pl.ds(r0, _C), 0:5]
            d2 = jax.lax.dot_general(
                ai, bj, (((1,), (0,)), ((), ())),
                preferred_element_type=jnp.float32,
            )
            d2 = jnp.maximum(d2, 1e-12)
            dist = d2 * jax.lax.rsqrt(d2)
            mi = row_ref[0, pl.ds(r0, _C), 5:6]
            w = adj_ref[0, pl.ds(c * _C, _C), :].astype(jnp.float32) * dist * mi
            return t1 + jnp.sum(w, axis=0, keepdims=True)

        return chunk

    t1 = jax.lax.fori_loop(
        0, H // _C, make_chunk(adj_top_ref, 0), jnp.zeros((1, N), jnp.float32)
    )
    t1 = jax.lax.fori_loop(0, H // _C, make_chunk(adj_bot_ref, H), t1)
    pair = jnp.sum(t1 * colm)

    ids = row_ref[0, :, 6:7]
    sp = jax.lax.broadcasted_iota(jnp.int32, (1, _SP), 1).astype(jnp.float32)
    oh = (ids == sp).astype(jnp.float32)
    atom = jnp.sum(oh * se_ref[0])

    out_ref[...] = jnp.full_like(out_ref, atom + pair)


def kernel(node_indices, positions, adjacency, mask, species_energy, pair_weight):
    B, N = node_indices.shape
    S = species_energy.shape[0]

    maskf = mask.astype(jnp.float32)
    idsf = jnp.where(mask, node_indices, -1).astype(jnp.float32)
    r2 = jnp.sum(positions * positions, axis=-1, keepdims=True)
    onesc = jnp.ones_like(r2)

    rowpack = jnp.concatenate(
        [positions, r2, onesc, maskf[:, :, None], idsf[:, :, None]], axis=-1
    )
    rowpack = jnp.pad(rowpack, ((0, 0), (0, 0), (0, 128 - 7)))

    pw = pair_weight.astype(jnp.float32)
    colpack = jnp.concatenate(
        [
            -2.0 * positions.transpose(0, 2, 1),
            onesc.transpose(0, 2, 1),
            r2.transpose(0, 2, 1) + 2e-4,
            pw * maskf[:, None, :],
            jnp.zeros((B, 2, N), jnp.float32),
        ],
        axis=1,
    )

    se_row = jnp.zeros((1, 1, _SP), jnp.float32).at[0, 0, :S].set(species_energy)

    out = pl.pallas_call(
        _body,
        grid=(B,),
        in_specs=[
            pl.BlockSpec((1, N, 128), lambda b: (b, 0, 0)),
            pl.BlockSpec((1, 8, N), lambda b: (b, 0, 0)),
            pl.BlockSpec((1, N // 2, N), lambda b: (b, 0, 0)),
            pl.BlockSpec((1, N // 2, N), lambda b: (b, 1, 0)),
            pl.BlockSpec((1, 1, _SP), lambda b: (0, 0, 0)),
        ],
        out_specs=pl.BlockSpec((1, 1, 128), lambda b: (b, 0, 0)),
        out_shape=jax.ShapeDtypeStruct((B, 1, 128), jnp.float32),
    )(rowpack, colpack, adjacency, adjacency, se_row)

    return out[:, 0, 0]


# speedup vs baseline: 2.0675x; 1.0139x over previous
import jax
import jax.numpy as jnp
from jax.experimental import pallas as pl

_C = 512
_SP = 128


def _body(row_ref, col_ref, adj_ref, se_ref, out_ref):
    N = adj_ref.shape[2]
    bj = col_ref[0, 0:5, :]
    colm = col_ref[0, 5:6, :]

    def chunk(c, t1):
        ai = row_ref[0, pl.ds(c * _C, _C), 0:5]
        d2 = jax.lax.dot_general(
            ai, bj, (((1,), (0,)), ((), ())),
            preferred_element_type=jnp.float32,
        )
        d2 = jnp.maximum(d2, 1e-12)
        dist = d2 * jax.lax.rsqrt(d2)
        w = adj_ref[0, pl.ds(c * _C, _C), :].astype(jnp.float32) * dist
        return t1 + jnp.sum(w, axis=0, keepdims=True)

    t1 = jax.lax.fori_loop(0, N // _C, chunk, jnp.zeros((1, N), jnp.float32))
    pair = jnp.sum(t1 * colm)

    ids = row_ref[0, :, 6:7]
    sp = jax.lax.broadcasted_iota(jnp.int32, (1, _SP), 1).astype(jnp.float32)
    oh = (ids == sp).astype(jnp.float32)
    atom = jnp.sum(oh * se_ref[0])

    out_ref[...] = jnp.full_like(out_ref, atom + pair)


def kernel(node_indices, positions, adjacency, mask, species_energy, pair_weight):
    B, N = node_indices.shape
    S = species_energy.shape[0]

    maskf = mask.astype(jnp.float32)
    mcol = maskf[:, :, None]
    idsf = jnp.where(mask, node_indices, -1).astype(jnp.float32)
    r2 = jnp.sum(positions * positions, axis=-1, keepdims=True)

    rowpack = jnp.concatenate(
        [positions * mcol, r2 * mcol, mcol, mcol, idsf[:, :, None]], axis=-1
    )
    rowpack = jnp.pad(rowpack, ((0, 0), (0, 0), (0, 128 - 7)))

    pw = pair_weight.astype(jnp.float32)
    colpack = jnp.concatenate(
        [
            -2.0 * positions.transpose(0, 2, 1),
            jnp.ones((B, 1, N), jnp.float32),
            r2.transpose(0, 2, 1) + 2e-4,
            pw * maskf[:, None, :],
            jnp.zeros((B, 2, N), jnp.float32),
        ],
        axis=1,
    )

    se_row = jnp.zeros((1, 1, _SP), jnp.float32).at[0, 0, :S].set(species_energy)

    out = pl.pallas_call(
        _body,
        grid=(B,),
        in_specs=[
            pl.BlockSpec((1, N, 128), lambda b: (b, 0, 0)),
            pl.BlockSpec((1, 8, N), lambda b: (b, 0, 0)),
            pl.BlockSpec((1, N, N), lambda b: (b, 0, 0)),
            pl.BlockSpec((1, 1, _SP), lambda b: (0, 0, 0)),
        ],
        out_specs=pl.BlockSpec((1, 1, 128), lambda b: (b, 0, 0)),
        out_shape=jax.ShapeDtypeStruct((B, 1, 128), jnp.float32),
    )(rowpack, colpack, adjacency, se_row)

    return out[:, 0, 0]


# speedup vs baseline: 2.1911x; 1.0598x over previous
import jax
import jax.numpy as jnp
from jax.experimental import pallas as pl
from jax.experimental.pallas import tpu as pltpu

_C = 512
_Q = 4
_SP = 128


def _body(row_ref, col_ref, adj_ref, se_ref, out_ref, abuf, sems):
    B = adj_ref.shape[0]
    N = adj_ref.shape[2]
    nc = N // _C
    b = pl.program_id(0)

    def start_copy(g, slot):
        gb = g // nc
        gc = jax.lax.rem(g, nc)
        pltpu.make_async_copy(
            adj_ref.at[gb, pl.ds(gc * _C, _C), :], abuf.at[slot], sems.at[slot]
        ).start()

    @pl.when(b == 0)
    def _():
        for q in range(_Q):
            start_copy(q, q)

    bj = col_ref[0, 0:5, :]
    colm = col_ref[0, 5:6, :]

    def chunk(c, t1):
        g = b * nc + c
        slot = jax.lax.rem(g, _Q)
        pltpu.make_async_copy(
            adj_ref.at[b, pl.ds(c * _C, _C), :], abuf.at[slot], sems.at[slot]
        ).wait()
        ai = row_ref[0, pl.ds(c * _C, _C), 0:5]
        d2 = jax.lax.dot_general(
            ai, bj, (((1,), (0,)), ((), ())),
            preferred_element_type=jnp.float32,
        )
        d2 = jnp.maximum(d2, 1e-12)
        dist = d2 * jax.lax.rsqrt(d2)
        w = abuf[slot].astype(jnp.float32) * dist
        t1 = t1 + jnp.sum(w, axis=0, keepdims=True)

        @pl.when(g + _Q < B * nc)
        def _():
            start_copy(g + _Q, slot)

        return t1

    t1 = jax.lax.fori_loop(0, nc, chunk, jnp.zeros((1, N), jnp.float32))
    pair = jnp.sum(t1 * colm)

    ids = row_ref[0, :, 6:7]
    sp = jax.lax.broadcasted_iota(jnp.int32, (1, _SP), 1).astype(jnp.float32)
    oh = (ids == sp).astype(jnp.float32)
    atom = jnp.sum(oh * se_ref[0])

    out_ref[...] = jnp.full_like(out_ref, atom + pair)


def kernel(node_indices, positions, adjacency, mask, species_energy, pair_weight):
    B, N = node_indices.shape
    S = species_energy.shape[0]

    maskf = mask.astype(jnp.float32)
    mcol = maskf[:, :, None]
    idsf = jnp.where(mask, node_indices, -1).astype(jnp.float32)
    r2 = jnp.sum(positions * positions, axis=-1, keepdims=True)

    rowpack = jnp.concatenate(
        [positions * mcol, r2 * mcol, mcol, mcol, idsf[:, :, None]], axis=-1
    )
    rowpack = jnp.pad(rowpack, ((0, 0), (0, 0), (0, 128 - 7)))

    pw = pair_weight.astype(jnp.float32)
    colpack = jnp.concatenate(
        [
            -2.0 * positions.transpose(0, 2, 1),
            jnp.ones((B, 1, N), jnp.float32),
            r2.transpose(0, 2, 1) + 2e-4,
            pw * maskf[:, None, :],
            jnp.zeros((B, 2, N), jnp.float32),
        ],
        axis=1,
    )

    se_row = jnp.zeros((1, 1, _SP), jnp.float32).at[0, 0, :S].set(species_energy)

    out = pl.pallas_call(
        _body,
        grid=(B,),
        in_specs=[
            pl.BlockSpec((1, N, 128), lambda b: (b, 0, 0)),
            pl.BlockSpec((1, 8, N), lambda b: (b, 0, 0)),
            pl.BlockSpec(memory_space=pl.ANY),
            pl.BlockSpec((1, 1, _SP), lambda b: (0, 0, 0)),
        ],
        out_specs=pl.BlockSpec((1, 1, 128), lambda b: (b, 0, 0)),
        out_shape=jax.ShapeDtypeStruct((B, 1, 128), jnp.float32),
        scratch_shapes=[
            pltpu.VMEM((_Q, _C, N), jnp.int32),
            pltpu.SemaphoreType.DMA((_Q,)),
        ],
    )(rowpack, colpack, adjacency, se_row)

    return out[:, 0, 0]
